# Initial kernel scaffold; baseline (speedup 1.0000x reference)
import jax
import jax.numpy as jnp
from jax.experimental import pallas as pl


def kernel(pressure, flow, acoustic, edge_index, edge_attr, missing_mask, params):
    raise NotImplementedError("write your pallas kernel here")


# speedup vs baseline: 12.4013x; 12.4013x over previous
import functools

import jax
import jax.numpy as jnp
from jax import lax
from jax.experimental import pallas as pl
from jax.experimental.pallas import tpu as pltpu
from jax.experimental.pallas import tpu_sc as plsc

_B, _N, _E = 2, 1024, 16384
_H, _NH, _HD, _ED = 64, 4, 16, 2
_BN = _B * _N
_NSUB = 16
_EPS = _E // _NSUB
_CK = 128
_NCK = _EPS // _CK
_RPS = _N // _NSUB
_PREC = lax.Precision.DEFAULT


def _ln(x, g, b):
    m = jnp.mean(x, axis=-1, keepdims=True)
    v = jnp.mean((x - m) ** 2, axis=-1, keepdims=True)
    return (x - m) / jnp.sqrt(v + 1e-5) * g + b


def _dot(a, b):
    return lax.dot_general(a, b, (((1,), (0,)), ((), ())), precision=_PREC)


def _dot_hi(a, b):
    return lax.dot_general(a, b, (((1,), (0,)), ((), ())),
                           precision=lax.Precision.HIGHEST)


def _enc_body(xpad, ew1, eb1, eg1, ebb1, ew2, eb2, eg2, ebb2, eemb,
              qin, kvin):
    encs = []
    for i in range(3):
        h = jnp.maximum(_dot(xpad[i], ew1[i]) + eb1[i], 0.0)
        h = _ln(h, eg1[i], ebb1[i])
        h = jnp.maximum(_dot(h, ew2[i]) + eb2[i], 0.0)
        h = _ln(h, eg2[i], ebb2[i])
        encs.append(h + eemb[i])
    pe, fe, ae = encs
    for j, t in enumerate([pe, pe, fe, fe, ae, ae]):
        qin[j] = t
    for j, t in enumerate([fe, ae, pe, ae, pe, fe]):
        kvin[j] = t


def _eap_body(ea, mw, mb, eap):
    x = ea[...]
    w = mw[0]
    acc = jnp.broadcast_to(mb[0], (x.shape[0], w.shape[1]))
    for k in range(_ED):
        acc = acc + x[:, k:k + 1] * w[k:k + 1, :]
    eap[0] = acc


def _attn_body(qin, kvin, wq, bq, wk, bk, wv, bv, wo, bo, lg, lb, out):
    qm = qin[0]
    kvm = kvin[0]
    q = _dot(qm, wq[0]) + bq[0]
    k = _dot(kvm, wk[0]) + bk[0]
    v = _dot(kvm, wv[0]) + bv[0]
    heads = []
    for h in range(_NH):
        sl = slice(h * _HD, (h + 1) * _HD)
        qh, kh, vh = q[:, sl], k[:, sl], v[:, sl]
        s = lax.dot_general(qh, kh, (((1,), (1,)), ((), ())),
                            precision=_PREC) * 0.25
        s = s - jnp.max(s, axis=-1, keepdims=True)
        e = jnp.exp(s)
        p = e / jnp.sum(e, axis=-1, keepdims=True)
        heads.append(_dot(p, vh))
    att = jnp.concatenate(heads, axis=-1)
    o = _dot(att, wo[0]) + bo[0] + qm
    out[0] = _ln(o, lg[0], lb[0])


def _fuse_body(att, mi, wf, bf, lg, lb, h0, haug):
    m = mi[...]
    e = jnp.exp(m - jnp.max(m, axis=-1, keepdims=True))
    w = e / jnp.sum(e, axis=-1, keepdims=True)
    p_enh = (att[0] + att[1]) * 0.5
    f_enh = (att[2] + att[3]) * 0.5
    a_enh = (att[4] + att[5]) * 0.5
    fused = jnp.concatenate(
        [p_enh * w[:, 0:1], f_enh * w[:, 1:2], a_enh * w[:, 2:3]], axis=-1)
    h = jnp.maximum(_dot(fused, wf[...]) + bf[...], 0.0)
    hh = _ln(h, lg[...], lb[...])
    h0[...] = hh
    haug[...] = jnp.concatenate(
        [hh, jnp.ones((_BN, 1), jnp.float32),
         jnp.zeros((_BN, _H - 1), jnp.float32)], axis=-1)


def _impmlp_body(h0, neigh, maskf, il1, ib1, il2, ib2, w1sd,
                 h1, psd):
    x = h0[...]
    ctx = jnp.concatenate([neigh[...], x], axis=-1)
    imp = _dot(jnp.maximum(_dot(ctx, il1[...]) + ib1[...], 0.0),
               il2[...]) + ib2[...]
    mk = maskf[...]
    h = mk * imp + (1.0 - mk) * x
    h1[...] = h
    psd[...] = _dot(h, w1sd[...])


def _post_body(hprev, aggpre, cnt, m2, m2b, u1, u1b, lg, lb, w1sd,
               hn, psd):
    x = hprev[...]
    ccol = cnt[...]
    c2 = jnp.concatenate([ccol, ccol], axis=0)
    agg = _dot(aggpre[...], m2[...]) + c2 * m2b[...]
    u = jnp.maximum(_dot(jnp.concatenate([x, agg], axis=-1), u1[...])
                    + u1b[...], 0.0)
    h = x + _ln(u, lg[...], lb[...])
    hn[...] = h
    psd[...] = _dot(h, w1sd[...])


def _final_body(hprev, aggpre, cnt, m2, m2b, u1, u1b, lg, lb,
                hw1, hb1, hw2, hb2, y):
    x = hprev[...]
    ccol = cnt[...]
    c2 = jnp.concatenate([ccol, ccol], axis=0)
    agg = _dot(aggpre[...], m2[...]) + c2 * m2b[...]
    u = jnp.maximum(_dot(jnp.concatenate([x, agg], axis=-1), u1[...])
                    + u1b[...], 0.0)
    h = x + _ln(u, lg[...], lb[...])
    y[...] = _dot(jnp.maximum(_dot(h, hw1[...]) + hb1[...], 0.0),
                  hw2[...]) + hb2[...]


def _f32(shape):
    return jax.ShapeDtypeStruct(shape, jnp.float32)


def _mesh():
    return plsc.VectorSubcoreMesh(core_axis_name="c", subcore_axis_name="s")


_W = 2 * _H


def _sc_impute(h0aug, srcb, dst):
    @functools.partial(
        pl.kernel,
        out_type=_f32((_B, _N, _W)),
        mesh=_mesh(),
        scratch_types=[
            pltpu.VMEM((_CK,), jnp.int32),
            pltpu.VMEM((_CK,), jnp.int32),
            pltpu.VMEM((_CK, _W), jnp.float32),
            pltpu.VMEM((_RPS, _W), jnp.float32),
            pltpu.VMEM_SHARED((_N, _W), jnp.float32),
            pltpu.SemaphoreType.DMA,
        ],
    )
    def k(h0_h, srcb_h, dst_h, out_h, ixs, ixd, rows, zbuf, acc, sem):
        c = lax.axis_index("c")
        s = lax.axis_index("s")

        @pl.loop(0, _RPS)
        def _(j):
            for g in range(_W // 16):
                zbuf[j, pl.ds(g * 16, 16)] = jnp.zeros((16,), jnp.float32)

        row0 = pl.multiple_of(s * _RPS, 8)
        pltpu.sync_copy(zbuf, acc.at[pl.ds(row0, _RPS)])
        plsc.subcore_barrier()

        @pl.loop(0, _NCK)
        def _(i):
            base = pl.multiple_of(c * _E + s * _EPS + i * _CK, 8)
            eb = pl.multiple_of(s * _EPS + i * _CK, 8)
            pltpu.sync_copy(srcb_h.at[pl.ds(base, _CK)], ixs)
            pltpu.sync_copy(dst_h.at[pl.ds(eb, _CK)], ixd)
            pltpu.async_copy(h0_h.at[ixs], rows, sem).wait()
            pltpu.sync_copy(rows, acc.at[ixd], add=True)

        plsc.subcore_barrier()
        pltpu.sync_copy(acc.at[pl.ds(row0, _RPS)],
                        out_h.at[c, pl.ds(row0, _RPS)])

    return k(h0aug, srcb, dst)


def _sc_mp(psd, eap, srcb, dstb, dst):
    @functools.partial(
        pl.kernel,
        out_type=_f32((_B, _N, _W)),
        mesh=_mesh(),
        scratch_types=[
            pltpu.VMEM((_CK,), jnp.int32),
            pltpu.VMEM((_CK,), jnp.int32),
            pltpu.VMEM((_CK,), jnp.int32),
            pltpu.VMEM((_CK, _W), jnp.float32),
            pltpu.VMEM((_CK, _W), jnp.float32),
            pltpu.VMEM((_CK, _W), jnp.float32),
            pltpu.VMEM((_CK, _W), jnp.float32),
            pltpu.VMEM((_RPS, _W), jnp.float32),
            pltpu.VMEM_SHARED((_N, _W), jnp.float32),
            pltpu.SemaphoreType.DMA,
            pltpu.SemaphoreType.DMA,
            pltpu.SemaphoreType.DMA,
        ],
    )
    def k(psd_h, eap_h, srcb_h, dstb_h, dst_h, out_h,
          ixs, ixdg, ixd, rs, rd, re, rr, zbuf, acc, sem1, sem2, sem3):
        c = lax.axis_index("c")
        s = lax.axis_index("s")

        @pl.loop(0, _RPS)
        def _(j):
            for g in range(_W // 16):
                zbuf[j, pl.ds(g * 16, 16)] = jnp.zeros((16,), jnp.float32)

        @pl.loop(0, _CK)
        def _(j):
            for g in range(_H // 16, _W // 16):
                rr[j, pl.ds(g * 16, 16)] = jnp.zeros((16,), jnp.float32)

        row0 = pl.multiple_of(s * _RPS, 8)
        pltpu.sync_copy(zbuf, acc.at[pl.ds(row0, _RPS)])
        plsc.subcore_barrier()

        @pl.loop(0, _NCK)
        def _(i):
            eb = pl.multiple_of(s * _EPS + i * _CK, 8)
            base = pl.multiple_of(c * _E + s * _EPS + i * _CK, 8)
            pltpu.sync_copy(srcb_h.at[pl.ds(base, _CK)], ixs)
            pltpu.sync_copy(dstb_h.at[pl.ds(base, _CK)], ixdg)
            pltpu.sync_copy(dst_h.at[pl.ds(eb, _CK)], ixd)
            cp1 = pltpu.async_copy(psd_h.at[ixs], rs, sem1)
            cp2 = pltpu.async_copy(psd_h.at[ixdg], rd, sem2)
            cp3 = pltpu.async_copy(eap_h.at[pl.ds(eb, _CK)], re, sem3)
            cp1.wait()
            cp2.wait()
            cp3.wait()

            @pl.loop(0, _CK)
            def _(j):
                for g in range(_H // 16):
                    sl = pl.ds(g * 16, 16)
                    v = (rs[j, sl] + rd[j, pl.ds(_H + g * 16, 16)]
                         + re[j, sl])
                    rr[j, sl] = jnp.maximum(v, 0.0)

            pltpu.sync_copy(rr, acc.at[ixd], add=True)

        plsc.subcore_barrier()
        pltpu.sync_copy(acc.at[pl.ds(row0, _RPS)],
                        out_h.at[c, pl.ds(row0, _RPS)])

    return k(psd, eap, srcb, dstb, dst)


def kernel(pressure, flow, acoustic, edge_index, edge_attr, missing_mask,
           params):
    f32 = jnp.float32

    def pad8(x):
        return jnp.pad(x.astype(f32), ((0, 0), (0, 8 - x.shape[1])))

    xpad = jnp.stack([pad8(pressure.reshape(_BN, -1)),
                      pad8(flow.reshape(_BN, -1)),
                      pad8(acoustic.reshape(_BN, -1))])

    encs = [params['p_enc'], params['f_enc'], params['a_enc']]
    ew1 = jnp.stack([jnp.pad(p['l1']['w'], ((0, 8 - p['l1']['w'].shape[0]),
                                            (0, 0))) for p in encs])
    eb1 = jnp.stack([p['l1']['b'].reshape(1, _H) for p in encs])
    eg1 = jnp.stack([p['ln1']['g'].reshape(1, _H) for p in encs])
    ebb1 = jnp.stack([p['ln1']['b'].reshape(1, _H) for p in encs])
    ew2 = jnp.stack([p['l2']['w'] for p in encs])
    eb2 = jnp.stack([p['l2']['b'].reshape(1, _H) for p in encs])
    eg2 = jnp.stack([p['ln2']['g'].reshape(1, _H) for p in encs])
    ebb2 = jnp.stack([p['ln2']['b'].reshape(1, _H) for p in encs])
    eemb = jnp.stack([p['emb'].reshape(1, _H) for p in encs])

    mp = params['mp']
    mw = jnp.stack([jnp.pad(lp['m1']['w'][2 * _H:], ((0, 0), (0, _W - _H)))
                    for lp in mp])
    mb = jnp.stack([jnp.pad(lp['m1']['b'], (0, _W - _H)).reshape(1, _W)
                    for lp in mp])

    qin, kvin = pl.pallas_call(
        _enc_body,
        out_shape=[_f32((6, _BN, _H)), _f32((6, _BN, _H))],
    )(xpad, ew1, eb1, eg1, ebb1, ew2, eb2, eg2, ebb2, eemb)

    _EBLK = 2048
    eap3 = pl.pallas_call(
        _eap_body,
        grid=(3, _E // _EBLK),
        in_specs=[pl.BlockSpec((_EBLK, _ED), lambda l, e: (e, 0)),
                  pl.BlockSpec((1, _ED, _W), lambda l, e: (l, 0, 0)),
                  pl.BlockSpec((1, 1, _W), lambda l, e: (l, 0, 0))],
        out_specs=pl.BlockSpec((1, _EBLK, _W), lambda l, e: (l, e, 0)),
        out_shape=_f32((3, _E, _W)),
    )(edge_attr.astype(f32), mw, mb)

    at = params['attn']
    order = ['p2f', 'p2a', 'f2p', 'f2a', 'a2p', 'a2f']
    wq = jnp.stack([at[k]['q']['w'] for k in order])
    bq = jnp.stack([at[k]['q']['b'].reshape(1, _H) for k in order])
    wk = jnp.stack([at[k]['k']['w'] for k in order])
    bk = jnp.stack([at[k]['k']['b'].reshape(1, _H) for k in order])
    wv = jnp.stack([at[k]['v']['w'] for k in order])
    bv = jnp.stack([at[k]['v']['b'].reshape(1, _H) for k in order])
    wo = jnp.stack([at[k]['o']['w'] for k in order])
    bo = jnp.stack([at[k]['o']['b'].reshape(1, _H) for k in order])
    alg = jnp.stack([at[k]['ln']['g'].reshape(1, _H) for k in order])
    alb = jnp.stack([at[k]['ln']['b'].reshape(1, _H) for k in order])

    big = pl.BlockSpec((1, _N, _H), lambda l, b: (l, b, 0))
    wsp = pl.BlockSpec((1, _H, _H), lambda l, b: (l, 0, 0))
    bsp = pl.BlockSpec((1, 1, _H), lambda l, b: (l, 0, 0))
    att = pl.pallas_call(
        _attn_body,
        grid=(6, _B),
        in_specs=[big, big, wsp, bsp, wsp, bsp, wsp, bsp, wsp, bsp,
                  bsp, bsp],
        out_specs=big,
        out_shape=_f32((6, _BN, _H)),
    )(qin, kvin, wq, bq, wk, bk, wv, bv, wo, bo, alg, alb)

    fu = params['fusion']
    h0, h0aug = pl.pallas_call(
        _fuse_body,
        out_shape=[_f32((_BN, _H)), _f32((_BN, _W))],
    )(att, params['modal_imp'].reshape(1, 3), fu['lin']['w'],
      fu['lin']['b'].reshape(1, _H), fu['ln']['g'].reshape(1, _H),
      fu['ln']['b'].reshape(1, _H))

    ei = edge_index.astype(jnp.int32)
    src, dst = ei[0], ei[1]
    offs = (jnp.arange(_B, dtype=jnp.int32) * _N)[:, None]
    srcb = (src[None, :] + offs).reshape(-1)
    dstb = (dst[None, :] + offs).reshape(-1)

    def w1sd(lp):
        w = lp['m1']['w']
        return jnp.concatenate([w[:_H], w[_H:2 * _H]], axis=1)

    io = _sc_impute(h0aug, srcb, dst)
    neigh = io[:, :, :_H].reshape(_BN, _H)
    cnt = io[0, :, _H:_H + 1]

    imp = params['imp']
    h1, psd = pl.pallas_call(
        _impmlp_body,
        out_shape=[_f32((_BN, _H)), _f32((_BN, _W))],
    )(h0, neigh, missing_mask.reshape(_BN, 1).astype(f32),
      imp['l1']['w'], imp['l1']['b'].reshape(1, _H), imp['l2']['w'],
      imp['l2']['b'].reshape(1, _H), w1sd(mp[0]))

    h = h1
    for li in range(3):
        lp = mp[li]
        aggpre = _sc_mp(psd, eap3[li], srcb, dstb, dst)[:, :, :_H]
        if li < 2:
            h, psd = pl.pallas_call(
                _post_body,
                out_shape=[_f32((_BN, _H)), _f32((_BN, _W))],
            )(h, aggpre.reshape(_BN, _H), cnt, lp['m2']['w'],
              lp['m2']['b'].reshape(1, _H), lp['u1']['w'],
              lp['u1']['b'].reshape(1, _H), lp['ln']['g'].reshape(1, _H),
              lp['ln']['b'].reshape(1, _H), w1sd(mp[li + 1]))
        else:
            hd = params['head']
            y = pl.pallas_call(
                _final_body,
                out_shape=_f32((_BN, 1)),
            )(h, aggpre.reshape(_BN, _H), cnt, lp['m2']['w'],
              lp['m2']['b'].reshape(1, _H), lp['u1']['w'],
              lp['u1']['b'].reshape(1, _H), lp['ln']['g'].reshape(1, _H),
              lp['ln']['b'].reshape(1, _H), hd['l1']['w'],
              hd['l1']['b'].reshape(1, _H // 2), hd['l2']['w'],
              hd['l2']['b'].reshape(1, 1))

    return y.reshape(_B, _N, 1)


# speedup vs baseline: 12.4597x; 1.0047x over previous
import functools

import jax
import jax.numpy as jnp
from jax import lax
from jax.experimental import pallas as pl
from jax.experimental.pallas import tpu as pltpu
from jax.experimental.pallas import tpu_sc as plsc

_B, _N, _E = 2, 1024, 16384
_H, _NH, _HD, _ED = 64, 4, 16, 2
_BN = _B * _N
_NSUB = 16
_EPS = _E // _NSUB
_CK = 128
_NCK = _EPS // _CK
_RPS = _N // _NSUB
_PREC = lax.Precision.DEFAULT


def _ln(x, g, b):
    m = jnp.mean(x, axis=-1, keepdims=True)
    v = jnp.mean((x - m) ** 2, axis=-1, keepdims=True)
    return (x - m) / jnp.sqrt(v + 1e-5) * g + b


def _dot(a, b):
    return lax.dot_general(a, b, (((1,), (0,)), ((), ())), precision=_PREC)


def _dot_hi(a, b):
    return lax.dot_general(a, b, (((1,), (0,)), ((), ())),
                           precision=lax.Precision.HIGHEST)


def _enc_body(xpad, ew1, eb1, eg1, ebb1, ew2, eb2, eg2, ebb2, eemb,
              qin, kvin):
    encs = []
    for i in range(3):
        if i == 1:
            h = jnp.maximum(xpad[i][:, 0:1] * ew1[i][0:1, :] + eb1[i], 0.0)
        else:
            h = jnp.maximum(_dot(xpad[i], ew1[i]) + eb1[i], 0.0)
        h = _ln(h, eg1[i], ebb1[i])
        h = jnp.maximum(_dot(h, ew2[i]) + eb2[i], 0.0)
        h = _ln(h, eg2[i], ebb2[i])
        encs.append(h + eemb[i])
    pe, fe, ae = encs
    for j, t in enumerate([pe, pe, fe, fe, ae, ae]):
        qin[j] = t
    for j, t in enumerate([fe, ae, pe, ae, pe, fe]):
        kvin[j] = t


def _eap_body(ea, mw, mb, eap):
    x = ea[...].astype(jnp.bfloat16).astype(jnp.float32)
    w = mw[0].astype(jnp.bfloat16).astype(jnp.float32)
    acc = jnp.broadcast_to(mb[0], (x.shape[0], w.shape[1]))
    for k in range(_ED):
        acc = acc + x[:, k:k + 1] * w[k:k + 1, :]
    eap[0] = acc


def _attn_body(qin, kvin, wq, bq, wk, bk, wv, bv, wo, bo, lg, lb, out):
    qm = qin[0]
    kvm = kvin[0]
    q = _dot(qm, wq[0]) + bq[0]
    k = _dot(kvm, wk[0]) + bk[0]
    v = _dot(kvm, wv[0]) + bv[0]
    heads = []
    for h in range(_NH):
        sl = slice(h * _HD, (h + 1) * _HD)
        qh, kh, vh = q[:, sl], k[:, sl], v[:, sl]
        s = lax.dot_general(qh, kh, (((1,), (1,)), ((), ())),
                            precision=_PREC) * 0.25
        s = s - jnp.max(s, axis=-1, keepdims=True)
        e = jnp.exp(s)
        p = e / jnp.sum(e, axis=-1, keepdims=True)
        heads.append(_dot(p, vh))
    att = jnp.concatenate(heads, axis=-1)
    o = _dot(att, wo[0]) + bo[0] + qm
    out[0] = _ln(o, lg[0], lb[0])


def _fuse_body(att, mi, wf, bf, lg, lb, h0, haug):
    m = mi[...]
    e = jnp.exp(m - jnp.max(m, axis=-1, keepdims=True))
    w = e / jnp.sum(e, axis=-1, keepdims=True)
    p_enh = (att[0] + att[1]) * 0.5
    f_enh = (att[2] + att[3]) * 0.5
    a_enh = (att[4] + att[5]) * 0.5
    fused = jnp.concatenate(
        [p_enh * w[:, 0:1], f_enh * w[:, 1:2], a_enh * w[:, 2:3]], axis=-1)
    h = jnp.maximum(_dot(fused, wf[...]) + bf[...], 0.0)
    hh = _ln(h, lg[...], lb[...])
    h0[...] = hh
    haug[...] = jnp.concatenate(
        [hh, jnp.ones((_BN, 1), jnp.float32),
         jnp.zeros((_BN, _H - 1), jnp.float32)], axis=-1)


def _impmlp_body(h0, neigh, maskf, il1, ib1, il2, ib2, w1sd,
                 h1, psd):
    x = h0[...]
    ctx = jnp.concatenate([neigh[...], x], axis=-1)
    imp = _dot(jnp.maximum(_dot(ctx, il1[...]) + ib1[...], 0.0),
               il2[...]) + ib2[...]
    mk = maskf[...]
    h = mk * imp + (1.0 - mk) * x
    h1[...] = h
    psd[...] = _dot(h, w1sd[...])


def _post_body(hprev, aggpre, cnt, m2, m2b, u1, u1b, lg, lb, w1sd,
               hn, psd):
    x = hprev[...]
    ccol = cnt[...]
    c2 = jnp.concatenate([ccol, ccol], axis=0)
    agg = _dot(aggpre[...], m2[...]) + c2 * m2b[...]
    u = jnp.maximum(_dot(jnp.concatenate([x, agg], axis=-1), u1[...])
                    + u1b[...], 0.0)
    h = x + _ln(u, lg[...], lb[...])
    hn[...] = h
    psd[...] = _dot(h, w1sd[...])


def _final_body(hprev, aggpre, cnt, m2, m2b, u1, u1b, lg, lb,
                hw1, hb1, hw2, hb2, y):
    x = hprev[...]
    ccol = cnt[...]
    c2 = jnp.concatenate([ccol, ccol], axis=0)
    agg = _dot(aggpre[...], m2[...]) + c2 * m2b[...]
    u = jnp.maximum(_dot(jnp.concatenate([x, agg], axis=-1), u1[...])
                    + u1b[...], 0.0)
    h = x + _ln(u, lg[...], lb[...])
    y[...] = _dot(jnp.maximum(_dot(h, hw1[...]) + hb1[...], 0.0),
                  hw2[...]) + hb2[...]


def _f32(shape):
    return jax.ShapeDtypeStruct(shape, jnp.float32)


def _mesh():
    return plsc.VectorSubcoreMesh(core_axis_name="c", subcore_axis_name="s")


_W = 2 * _H


def _sc_impute(h0aug, srcb, dst):
    @functools.partial(
        pl.kernel,
        out_type=_f32((_B, _N, _W)),
        mesh=_mesh(),
        scratch_types=[
            pltpu.VMEM((_CK,), jnp.int32),
            pltpu.VMEM((_CK,), jnp.int32),
            pltpu.VMEM((_CK, _W), jnp.float32),
            pltpu.VMEM((_RPS, _W), jnp.float32),
            pltpu.VMEM_SHARED((_N, _W), jnp.float32),
            pltpu.SemaphoreType.DMA,
        ],
    )
    def k(h0_h, srcb_h, dst_h, out_h, ixs, ixd, rows, zbuf, acc, sem):
        c = lax.axis_index("c")
        s = lax.axis_index("s")

        @pl.loop(0, _RPS)
        def _(j):
            for g in range(_W // 16):
                zbuf[j, pl.ds(g * 16, 16)] = jnp.zeros((16,), jnp.float32)

        row0 = pl.multiple_of(s * _RPS, 8)
        pltpu.sync_copy(zbuf, acc.at[pl.ds(row0, _RPS)])
        plsc.subcore_barrier()

        @pl.loop(0, _NCK)
        def _(i):
            base = pl.multiple_of(c * _E + s * _EPS + i * _CK, 8)
            eb = pl.multiple_of(s * _EPS + i * _CK, 8)
            pltpu.sync_copy(srcb_h.at[pl.ds(base, _CK)], ixs)
            pltpu.sync_copy(dst_h.at[pl.ds(eb, _CK)], ixd)
            pltpu.async_copy(h0_h.at[ixs], rows, sem).wait()
            pltpu.sync_copy(rows, acc.at[ixd], add=True)

        plsc.subcore_barrier()
        pltpu.sync_copy(acc.at[pl.ds(row0, _RPS)],
                        out_h.at[c, pl.ds(row0, _RPS)])

    return k(h0aug, srcb, dst)


def _sc_mp(psd, eap, srcb, dstb, dst):
    @functools.partial(
        pl.kernel,
        out_type=_f32((_B, _N, _W)),
        mesh=_mesh(),
        scratch_types=[
            pltpu.VMEM((_CK,), jnp.int32),
            pltpu.VMEM((_CK,), jnp.int32),
            pltpu.VMEM((_CK,), jnp.int32),
            pltpu.VMEM((_CK, _W), jnp.float32),
            pltpu.VMEM((_CK, _W), jnp.float32),
            pltpu.VMEM((_CK, _W), jnp.float32),
            pltpu.VMEM((_CK, _W), jnp.float32),
            pltpu.VMEM((_RPS, _W), jnp.float32),
            pltpu.VMEM_SHARED((_N, _W), jnp.float32),
            pltpu.SemaphoreType.DMA,
            pltpu.SemaphoreType.DMA,
            pltpu.SemaphoreType.DMA,
        ],
    )
    def k(psd_h, eap_h, srcb_h, dstb_h, dst_h, out_h,
          ixs, ixdg, ixd, rs, rd, re, rr, zbuf, acc, sem1, sem2, sem3):
        c = lax.axis_index("c")
        s = lax.axis_index("s")

        @pl.loop(0, _RPS)
        def _(j):
            for g in range(_W // 16):
                zbuf[j, pl.ds(g * 16, 16)] = jnp.zeros((16,), jnp.float32)

        @pl.loop(0, _CK)
        def _(j):
            for g in range(_H // 16, _W // 16):
                rr[j, pl.ds(g * 16, 16)] = jnp.zeros((16,), jnp.float32)

        row0 = pl.multiple_of(s * _RPS, 8)
        pltpu.sync_copy(zbuf, acc.at[pl.ds(row0, _RPS)])
        plsc.subcore_barrier()

        @pl.loop(0, _NCK)
        def _(i):
            eb = pl.multiple_of(s * _EPS + i * _CK, 8)
            base = pl.multiple_of(c * _E + s * _EPS + i * _CK, 8)
            pltpu.sync_copy(srcb_h.at[pl.ds(base, _CK)], ixs)
            pltpu.sync_copy(dstb_h.at[pl.ds(base, _CK)], ixdg)
            pltpu.sync_copy(dst_h.at[pl.ds(eb, _CK)], ixd)
            cp1 = pltpu.async_copy(psd_h.at[ixs], rs, sem1)
            cp2 = pltpu.async_copy(psd_h.at[ixdg], rd, sem2)
            cp3 = pltpu.async_copy(eap_h.at[pl.ds(eb, _CK)], re, sem3)
            cp1.wait()
            cp2.wait()
            cp3.wait()

            @pl.loop(0, _CK)
            def _(j):
                for g in range(_H // 16):
                    sl = pl.ds(g * 16, 16)
                    v = (rs[j, sl] + rd[j, pl.ds(_H + g * 16, 16)]
                         + re[j, sl])
                    rr[j, sl] = jnp.maximum(v, 0.0)

            pltpu.sync_copy(rr, acc.at[ixd], add=True)

        plsc.subcore_barrier()
        pltpu.sync_copy(acc.at[pl.ds(row0, _RPS)],
                        out_h.at[c, pl.ds(row0, _RPS)])

    return k(psd, eap, srcb, dstb, dst)


def kernel(pressure, flow, acoustic, edge_index, edge_attr, missing_mask,
           params):
    f32 = jnp.float32

    def pad8(x):
        return jnp.pad(x.astype(f32), ((0, 0), (0, 8 - x.shape[1])))

    xpad = jnp.stack([pad8(pressure.reshape(_BN, -1)),
                      pad8(flow.reshape(_BN, -1)),
                      pad8(acoustic.reshape(_BN, -1))])

    encs = [params['p_enc'], params['f_enc'], params['a_enc']]
    ew1 = jnp.stack([jnp.pad(p['l1']['w'], ((0, 8 - p['l1']['w'].shape[0]),
                                            (0, 0))) for p in encs])
    eb1 = jnp.stack([p['l1']['b'].reshape(1, _H) for p in encs])
    eg1 = jnp.stack([p['ln1']['g'].reshape(1, _H) for p in encs])
    ebb1 = jnp.stack([p['ln1']['b'].reshape(1, _H) for p in encs])
    ew2 = jnp.stack([p['l2']['w'] for p in encs])
    eb2 = jnp.stack([p['l2']['b'].reshape(1, _H) for p in encs])
    eg2 = jnp.stack([p['ln2']['g'].reshape(1, _H) for p in encs])
    ebb2 = jnp.stack([p['ln2']['b'].reshape(1, _H) for p in encs])
    eemb = jnp.stack([p['emb'].reshape(1, _H) for p in encs])

    mp = params['mp']
    mw = jnp.stack([jnp.pad(lp['m1']['w'][2 * _H:], ((0, 0), (0, _W - _H)))
                    for lp in mp])
    mb = jnp.stack([jnp.pad(lp['m1']['b'], (0, _W - _H)).reshape(1, _W)
                    for lp in mp])

    qin, kvin = pl.pallas_call(
        _enc_body,
        out_shape=[_f32((6, _BN, _H)), _f32((6, _BN, _H))],
    )(xpad, ew1, eb1, eg1, ebb1, ew2, eb2, eg2, ebb2, eemb)

    _EBLK = 2048
    eap3 = pl.pallas_call(
        _eap_body,
        grid=(3, _E // _EBLK),
        in_specs=[pl.BlockSpec((_EBLK, _ED), lambda l, e: (e, 0)),
                  pl.BlockSpec((1, _ED, _W), lambda l, e: (l, 0, 0)),
                  pl.BlockSpec((1, 1, _W), lambda l, e: (l, 0, 0))],
        out_specs=pl.BlockSpec((1, _EBLK, _W), lambda l, e: (l, e, 0)),
        out_shape=_f32((3, _E, _W)),
    )(edge_attr.astype(f32), mw, mb)

    at = params['attn']
    order = ['p2f', 'p2a', 'f2p', 'f2a', 'a2p', 'a2f']
    wq = jnp.stack([at[k]['q']['w'] for k in order])
    bq = jnp.stack([at[k]['q']['b'].reshape(1, _H) for k in order])
    wk = jnp.stack([at[k]['k']['w'] for k in order])
    bk = jnp.stack([at[k]['k']['b'].reshape(1, _H) for k in order])
    wv = jnp.stack([at[k]['v']['w'] for k in order])
    bv = jnp.stack([at[k]['v']['b'].reshape(1, _H) for k in order])
    wo = jnp.stack([at[k]['o']['w'] for k in order])
    bo = jnp.stack([at[k]['o']['b'].reshape(1, _H) for k in order])
    alg = jnp.stack([at[k]['ln']['g'].reshape(1, _H) for k in order])
    alb = jnp.stack([at[k]['ln']['b'].reshape(1, _H) for k in order])

    big = pl.BlockSpec((1, _N, _H), lambda l, b: (l, b, 0))
    wsp = pl.BlockSpec((1, _H, _H), lambda l, b: (l, 0, 0))
    bsp = pl.BlockSpec((1, 1, _H), lambda l, b: (l, 0, 0))
    att = pl.pallas_call(
        _attn_body,
        grid=(6, _B),
        in_specs=[big, big, wsp, bsp, wsp, bsp, wsp, bsp, wsp, bsp,
                  bsp, bsp],
        out_specs=big,
        out_shape=_f32((6, _BN, _H)),
    )(qin, kvin, wq, bq, wk, bk, wv, bv, wo, bo, alg, alb)

    fu = params['fusion']
    h0, h0aug = pl.pallas_call(
        _fuse_body,
        out_shape=[_f32((_BN, _H)), _f32((_BN, _W))],
    )(att, params['modal_imp'].reshape(1, 3), fu['lin']['w'],
      fu['lin']['b'].reshape(1, _H), fu['ln']['g'].reshape(1, _H),
      fu['ln']['b'].reshape(1, _H))

    ei = edge_index.astype(jnp.int32)
    src, dst = ei[0], ei[1]
    offs = (jnp.arange(_B, dtype=jnp.int32) * _N)[:, None]
    srcb = (src[None, :] + offs).reshape(-1)
    dstb = (dst[None, :] + offs).reshape(-1)

    def w1sd(lp):
        w = lp['m1']['w']
        return jnp.concatenate([w[:_H], w[_H:2 * _H]], axis=1)

    io = _sc_impute(h0aug, srcb, dst)
    neigh = io[:, :, :_H].reshape(_BN, _H)
    cnt = io[0, :, _H:_H + 1]

    imp = params['imp']
    h1, psd = pl.pallas_call(
        _impmlp_body,
        out_shape=[_f32((_BN, _H)), _f32((_BN, _W))],
    )(h0, neigh, missing_mask.reshape(_BN, 1).astype(f32),
      imp['l1']['w'], imp['l1']['b'].reshape(1, _H), imp['l2']['w'],
      imp['l2']['b'].reshape(1, _H), w1sd(mp[0]))

    h = h1
    for li in range(3):
        lp = mp[li]
        aggpre = _sc_mp(psd, eap3[li], srcb, dstb, dst)[:, :, :_H]
        if li < 2:
            h, psd = pl.pallas_call(
                _post_body,
                out_shape=[_f32((_BN, _H)), _f32((_BN, _W))],
            )(h, aggpre.reshape(_BN, _H), cnt, lp['m2']['w'],
              lp['m2']['b'].reshape(1, _H), lp['u1']['w'],
              lp['u1']['b'].reshape(1, _H), lp['ln']['g'].reshape(1, _H),
              lp['ln']['b'].reshape(1, _H), w1sd(mp[li + 1]))
        else:
            hd = params['head']
            y = pl.pallas_call(
                _final_body,
                out_shape=_f32((_BN, 1)),
            )(h, aggpre.reshape(_BN, _H), cnt, lp['m2']['w'],
              lp['m2']['b'].reshape(1, _H), lp['u1']['w'],
              lp['u1']['b'].reshape(1, _H), lp['ln']['g'].reshape(1, _H),
              lp['ln']['b'].reshape(1, _H), hd['l1']['w'],
              hd['l1']['b'].reshape(1, _H // 2), hd['l2']['w'],
              hd['l2']['b'].reshape(1, 1))

    return y.reshape(_B, _N, 1)


# speedup vs baseline: 14.3443x; 1.1513x over previous
import functools

import jax
import jax.numpy as jnp
from jax import lax
from jax.experimental import pallas as pl
from jax.experimental.pallas import tpu as pltpu
from jax.experimental.pallas import tpu_sc as plsc

_B, _N, _E = 2, 1024, 16384
_H, _NH, _HD, _ED = 64, 4, 16, 2
_BN = _B * _N
_NSUB = 16
_EPS = _E // _NSUB
_CK = 128
_NCK = _EPS // _CK
_RPS = _N // _NSUB
_PREC = lax.Precision.DEFAULT


def _ln(x, g, b):
    m = jnp.mean(x, axis=-1, keepdims=True)
    v = jnp.mean((x - m) ** 2, axis=-1, keepdims=True)
    return (x - m) / jnp.sqrt(v + 1e-5) * g + b


def _dot(a, b):
    return lax.dot_general(a, b, (((1,), (0,)), ((), ())), precision=_PREC)


def _dot_hi(a, b):
    return lax.dot_general(a, b, (((1,), (0,)), ((), ())),
                           precision=lax.Precision.HIGHEST)


def _enc_body(xpad, ew1, eb1, eg1, ebb1, ew2, eb2, eg2, ebb2, eemb,
              qin, kvin):
    encs = []
    for i in range(3):
        if i == 1:
            h = jnp.maximum(xpad[i][:, 0:1] * ew1[i][0:1, :] + eb1[i], 0.0)
        else:
            h = jnp.maximum(_dot(xpad[i], ew1[i]) + eb1[i], 0.0)
        h = _ln(h, eg1[i], ebb1[i])
        h = jnp.maximum(_dot(h, ew2[i]) + eb2[i], 0.0)
        h = _ln(h, eg2[i], ebb2[i])
        encs.append(h + eemb[i])
    pe, fe, ae = encs
    for j, t in enumerate([pe, pe, fe, fe, ae, ae]):
        qin[j] = t
    for j, t in enumerate([fe, ae, pe, ae, pe, fe]):
        kvin[j] = t


def _eap_body(ea, mw, mb, eap):
    x = ea[...].astype(jnp.bfloat16).astype(jnp.float32)
    w = mw[0].astype(jnp.bfloat16).astype(jnp.float32)
    acc = jnp.broadcast_to(mb[0], (x.shape[0], w.shape[1]))
    for k in range(_ED):
        acc = acc + x[:, k:k + 1] * w[k:k + 1, :]
    eap[0] = acc


def _attn_body(qin, kvin, wq, bq, wk, bk, wv, bv, wo, bo, lg, lb, out):
    qm = qin[0]
    kvm = kvin[0]
    q = _dot(qm, wq[0]) + bq[0]
    k = _dot(kvm, wk[0]) + bk[0]
    v = _dot(kvm, wv[0]) + bv[0]
    heads = []
    for h in range(_NH):
        sl = slice(h * _HD, (h + 1) * _HD)
        qh, kh, vh = q[:, sl], k[:, sl], v[:, sl]
        s = lax.dot_general(qh, kh, (((1,), (1,)), ((), ())),
                            precision=_PREC) * 0.25
        s = s - jnp.max(s, axis=-1, keepdims=True)
        e = jnp.exp(s)
        p = e / jnp.sum(e, axis=-1, keepdims=True)
        heads.append(_dot(p, vh))
    att = jnp.concatenate(heads, axis=-1)
    o = _dot(att, wo[0]) + bo[0] + qm
    out[0] = _ln(o, lg[0], lb[0])


def _fuse_body(att, mi, wf, bf, lg, lb, h0, haug):
    m = mi[...]
    e = jnp.exp(m - jnp.max(m, axis=-1, keepdims=True))
    w = e / jnp.sum(e, axis=-1, keepdims=True)
    p_enh = (att[0] + att[1]) * 0.5
    f_enh = (att[2] + att[3]) * 0.5
    a_enh = (att[4] + att[5]) * 0.5
    fused = jnp.concatenate(
        [p_enh * w[:, 0:1], f_enh * w[:, 1:2], a_enh * w[:, 2:3]], axis=-1)
    h = jnp.maximum(_dot(fused, wf[...]) + bf[...], 0.0)
    hh = _ln(h, lg[...], lb[...])
    h0[...] = hh
    haug[...] = jnp.concatenate(
        [hh, jnp.ones((_BN, 1), jnp.float32),
         jnp.zeros((_BN, _H - 1), jnp.float32)], axis=-1)


def _impmlp_body(h0, neigh, maskf, il1, ib1, il2, ib2, w1sd,
                 h1, psd):
    x = h0[...]
    ctx = jnp.concatenate([neigh[...], x], axis=-1)
    imp = _dot(jnp.maximum(_dot(ctx, il1[...]) + ib1[...], 0.0),
               il2[...]) + ib2[...]
    mk = maskf[...]
    h = mk * imp + (1.0 - mk) * x
    h1[...] = h
    psd[...] = _dot(h, w1sd[...])


def _post_body(hprev, aggpre, cnt, m2, m2b, u1, u1b, lg, lb, w1sd,
               hn, psd):
    x = hprev[...]
    ccol = cnt[...]
    c2 = jnp.concatenate([ccol, ccol], axis=0)
    agg = _dot(aggpre[...], m2[...]) + c2 * m2b[...]
    u = jnp.maximum(_dot(jnp.concatenate([x, agg], axis=-1), u1[...])
                    + u1b[...], 0.0)
    h = x + _ln(u, lg[...], lb[...])
    hn[...] = h
    psd[...] = _dot(h, w1sd[...])


def _final_body(hprev, aggpre, cnt, m2, m2b, u1, u1b, lg, lb,
                hw1, hb1, hw2, hb2, y):
    x = hprev[...]
    ccol = cnt[...]
    c2 = jnp.concatenate([ccol, ccol], axis=0)
    agg = _dot(aggpre[...], m2[...]) + c2 * m2b[...]
    u = jnp.maximum(_dot(jnp.concatenate([x, agg], axis=-1), u1[...])
                    + u1b[...], 0.0)
    h = x + _ln(u, lg[...], lb[...])
    y[...] = _dot(jnp.maximum(_dot(h, hw1[...]) + hb1[...], 0.0),
                  hw2[...]) + hb2[...]


def _f32(shape):
    return jax.ShapeDtypeStruct(shape, jnp.float32)


def _mesh():
    return plsc.VectorSubcoreMesh(core_axis_name="c", subcore_axis_name="s")


_W = 2 * _H


def _sc_impute(h0aug, srcb, dst):
    @functools.partial(
        pl.kernel,
        out_type=_f32((_B, _N, _W)),
        mesh=_mesh(),
        scratch_types=[
            pltpu.VMEM((_CK,), jnp.int32),
            pltpu.VMEM((_CK,), jnp.int32),
            pltpu.VMEM((_CK, _W), jnp.float32),
            pltpu.VMEM((_RPS, _W), jnp.float32),
            pltpu.VMEM_SHARED((_N, _W), jnp.float32),
            pltpu.SemaphoreType.DMA,
        ],
    )
    def k(h0_h, srcb_h, dst_h, out_h, ixs, ixd, rows, zbuf, acc, sem):
        c = lax.axis_index("c")
        s = lax.axis_index("s")

        @pl.loop(0, _RPS)
        def _(j):
            for g in range(_W // 16):
                zbuf[j, pl.ds(g * 16, 16)] = jnp.zeros((16,), jnp.float32)

        row0 = pl.multiple_of(s * _RPS, 8)
        pltpu.sync_copy(zbuf, acc.at[pl.ds(row0, _RPS)])
        plsc.subcore_barrier()

        @pl.loop(0, _NCK)
        def _(i):
            base = pl.multiple_of(c * _E + s * _EPS + i * _CK, 8)
            eb = pl.multiple_of(s * _EPS + i * _CK, 8)
            pltpu.sync_copy(srcb_h.at[pl.ds(base, _CK)], ixs)
            pltpu.sync_copy(dst_h.at[pl.ds(eb, _CK)], ixd)
            pltpu.async_copy(h0_h.at[ixs], rows, sem).wait()
            pltpu.sync_copy(rows, acc.at[ixd], add=True)

        plsc.subcore_barrier()
        pltpu.sync_copy(acc.at[pl.ds(row0, _RPS)],
                        out_h.at[c, pl.ds(row0, _RPS)])

    return k(h0aug, srcb, dst)


def _sc_mp(psd, eap, srcb2, dstb2, dst2):
    @functools.partial(
        pl.kernel,
        out_type=_f32((_B, _N, _W)),
        mesh=_mesh(),
        scratch_types=[
            pltpu.VMEM((_NCK, _CK), jnp.int32),
            pltpu.VMEM((_NCK, _CK), jnp.int32),
            pltpu.VMEM((_NCK, _CK), jnp.int32),
            pltpu.VMEM((_CK, _W), jnp.float32),
            pltpu.VMEM((_CK, _W), jnp.float32),
            pltpu.VMEM((_CK, _W), jnp.float32),
            pltpu.VMEM((_CK, _W), jnp.float32),
            pltpu.VMEM((_CK, _W), jnp.float32),
            pltpu.VMEM((_CK, _W), jnp.float32),
            pltpu.VMEM((_CK, _W), jnp.float32),
            pltpu.VMEM((8, _W), jnp.float32),
            pltpu.VMEM_SHARED((_N, _W), jnp.float32),
            pltpu.SemaphoreType.DMA,
            pltpu.SemaphoreType.DMA,
        ],
    )
    def k(psd_h, eap_h, srcb_h, dstb_h, dst_h, out_h,
          ixs, ixdg, ixd, rs0, rd0, re0, rs1, rd1, re1, rr, zbuf, acc,
          gsem0, gsem1):
        c = lax.axis_index("c")
        s = lax.axis_index("s")
        bufs = ((rs0, rd0, re0, gsem0), (rs1, rd1, re1, gsem1))

        @pl.loop(0, 8)
        def _(j):
            for g in range(_W // 16):
                zbuf[j, pl.ds(g * 16, 16)] = jnp.zeros((16,), jnp.float32)

        @pl.loop(0, _CK)
        def _(j):
            for g in range(_H // 16, _W // 16):
                rr[j, pl.ds(g * 16, 16)] = jnp.zeros((16,), jnp.float32)

        row0 = pl.multiple_of(s * _RPS, 8)

        @pl.loop(0, _RPS // 8)
        def _(t):
            pltpu.sync_copy(zbuf, acc.at[pl.ds(row0 + t * 8, 8)])

        ib0 = s * _NCK
        ib0b = c * (_E // _CK) + s * _NCK
        pltpu.sync_copy(srcb_h.at[pl.ds(ib0b, _NCK)], ixs)
        pltpu.sync_copy(dstb_h.at[pl.ds(ib0b, _NCK)], ixdg)
        pltpu.sync_copy(dst_h.at[pl.ds(ib0, _NCK)], ixd)
        plsc.subcore_barrier()

        def fire(i, bf):
            rs_, rd_, re_, gs_ = bf
            eb = pl.multiple_of(s * _EPS + i * _CK, 8)
            pltpu.async_copy(psd_h.at[ixs.at[i]], rs_, gs_)
            pltpu.async_copy(psd_h.at[ixdg.at[i]], rd_, gs_)
            pltpu.async_copy(eap_h.at[pl.ds(eb, _CK)], re_, gs_)

        def wait_gathers(bf):
            rs_, rd_, re_, gs_ = bf
            dummy = eap_h.at[pl.ds(0, _CK)]
            pltpu.make_async_copy(dummy, rs_, gs_).wait()
            pltpu.make_async_copy(dummy, rd_, gs_).wait()
            pltpu.make_async_copy(dummy, re_, gs_).wait()

        def work(i, bf):
            rs_, rd_, re_, gs_ = bf
            wait_gathers(bf)

            @pl.loop(0, _CK)
            def _(j):
                for g in range(_H // 16):
                    sl = pl.ds(g * 16, 16)
                    v = (rs_[j, sl] + rd_[j, pl.ds(_H + g * 16, 16)]
                         + re_[j, sl])
                    rr[j, sl] = jnp.maximum(v, 0.0)

            pltpu.sync_copy(rr, acc.at[ixd.at[i]], add=True)

        fire(0, bufs[0])

        @pl.loop(0, _NCK // 2)
        def _(t):
            i0 = t * 2
            fire(i0 + 1, bufs[1])
            work(i0, bufs[0])

            @pl.when(i0 + 2 < _NCK)
            def _():
                fire(i0 + 2, bufs[0])

            work(i0 + 1, bufs[1])

        plsc.subcore_barrier()
        pltpu.sync_copy(acc.at[pl.ds(row0, _RPS)],
                        out_h.at[c, pl.ds(row0, _RPS)])

    return k(psd, eap, srcb2, dstb2, dst2)


def kernel(pressure, flow, acoustic, edge_index, edge_attr, missing_mask,
           params):
    f32 = jnp.float32

    def pad8(x):
        return jnp.pad(x.astype(f32), ((0, 0), (0, 8 - x.shape[1])))

    xpad = jnp.stack([pad8(pressure.reshape(_BN, -1)),
                      pad8(flow.reshape(_BN, -1)),
                      pad8(acoustic.reshape(_BN, -1))])

    encs = [params['p_enc'], params['f_enc'], params['a_enc']]
    ew1 = jnp.stack([jnp.pad(p['l1']['w'], ((0, 8 - p['l1']['w'].shape[0]),
                                            (0, 0))) for p in encs])
    eb1 = jnp.stack([p['l1']['b'].reshape(1, _H) for p in encs])
    eg1 = jnp.stack([p['ln1']['g'].reshape(1, _H) for p in encs])
    ebb1 = jnp.stack([p['ln1']['b'].reshape(1, _H) for p in encs])
    ew2 = jnp.stack([p['l2']['w'] for p in encs])
    eb2 = jnp.stack([p['l2']['b'].reshape(1, _H) for p in encs])
    eg2 = jnp.stack([p['ln2']['g'].reshape(1, _H) for p in encs])
    ebb2 = jnp.stack([p['ln2']['b'].reshape(1, _H) for p in encs])
    eemb = jnp.stack([p['emb'].reshape(1, _H) for p in encs])

    mp = params['mp']
    mw = jnp.stack([jnp.pad(lp['m1']['w'][2 * _H:], ((0, 0), (0, _W - _H)))
                    for lp in mp])
    mb = jnp.stack([jnp.pad(lp['m1']['b'], (0, _W - _H)).reshape(1, _W)
                    for lp in mp])

    qin, kvin = pl.pallas_call(
        _enc_body,
        out_shape=[_f32((6, _BN, _H)), _f32((6, _BN, _H))],
    )(xpad, ew1, eb1, eg1, ebb1, ew2, eb2, eg2, ebb2, eemb)

    _EBLK = 2048
    eap3 = pl.pallas_call(
        _eap_body,
        grid=(3, _E // _EBLK),
        in_specs=[pl.BlockSpec((_EBLK, _ED), lambda l, e: (e, 0)),
                  pl.BlockSpec((1, _ED, _W), lambda l, e: (l, 0, 0)),
                  pl.BlockSpec((1, 1, _W), lambda l, e: (l, 0, 0))],
        out_specs=pl.BlockSpec((1, _EBLK, _W), lambda l, e: (l, e, 0)),
        out_shape=_f32((3, _E, _W)),
    )(edge_attr.astype(f32), mw, mb)

    at = params['attn']
    order = ['p2f', 'p2a', 'f2p', 'f2a', 'a2p', 'a2f']
    wq = jnp.stack([at[k]['q']['w'] for k in order])
    bq = jnp.stack([at[k]['q']['b'].reshape(1, _H) for k in order])
    wk = jnp.stack([at[k]['k']['w'] for k in order])
    bk = jnp.stack([at[k]['k']['b'].reshape(1, _H) for k in order])
    wv = jnp.stack([at[k]['v']['w'] for k in order])
    bv = jnp.stack([at[k]['v']['b'].reshape(1, _H) for k in order])
    wo = jnp.stack([at[k]['o']['w'] for k in order])
    bo = jnp.stack([at[k]['o']['b'].reshape(1, _H) for k in order])
    alg = jnp.stack([at[k]['ln']['g'].reshape(1, _H) for k in order])
    alb = jnp.stack([at[k]['ln']['b'].reshape(1, _H) for k in order])

    big = pl.BlockSpec((1, _N, _H), lambda l, b: (l, b, 0))
    wsp = pl.BlockSpec((1, _H, _H), lambda l, b: (l, 0, 0))
    bsp = pl.BlockSpec((1, 1, _H), lambda l, b: (l, 0, 0))
    att = pl.pallas_call(
        _attn_body,
        grid=(6, _B),
        in_specs=[big, big, wsp, bsp, wsp, bsp, wsp, bsp, wsp, bsp,
                  bsp, bsp],
        out_specs=big,
        out_shape=_f32((6, _BN, _H)),
    )(qin, kvin, wq, bq, wk, bk, wv, bv, wo, bo, alg, alb)

    fu = params['fusion']
    h0, h0aug = pl.pallas_call(
        _fuse_body,
        out_shape=[_f32((_BN, _H)), _f32((_BN, _W))],
    )(att, params['modal_imp'].reshape(1, 3), fu['lin']['w'],
      fu['lin']['b'].reshape(1, _H), fu['ln']['g'].reshape(1, _H),
      fu['ln']['b'].reshape(1, _H))

    ei = edge_index.astype(jnp.int32)
    src, dst = ei[0], ei[1]
    offs = (jnp.arange(_B, dtype=jnp.int32) * _N)[:, None]
    srcb = (src[None, :] + offs).reshape(-1)
    dstbf = (dst[None, :] + offs).reshape(-1)
    srcb2 = srcb.reshape(_B * _E // _CK, _CK)
    dstb2 = dstbf.reshape(_B * _E // _CK, _CK)
    dst2 = dst.reshape(_E // _CK, _CK)

    def w1sd(lp):
        w = lp['m1']['w']
        return jnp.concatenate([w[:_H], w[_H:2 * _H]], axis=1)

    io = _sc_impute(h0aug, srcb, dst)
    neigh = io[:, :, :_H].reshape(_BN, _H)
    cnt = io[0, :, _H:_H + 1]

    imp = params['imp']
    h1, psd = pl.pallas_call(
        _impmlp_body,
        out_shape=[_f32((_BN, _H)), _f32((_BN, _W))],
    )(h0, neigh, missing_mask.reshape(_BN, 1).astype(f32),
      imp['l1']['w'], imp['l1']['b'].reshape(1, _H), imp['l2']['w'],
      imp['l2']['b'].reshape(1, _H), w1sd(mp[0]))

    h = h1
    for li in range(3):
        lp = mp[li]
        aggpre = _sc_mp(psd, eap3[li], srcb2, dstb2, dst2)[:, :, :_H]
        if li < 2:
            h, psd = pl.pallas_call(
                _post_body,
                out_shape=[_f32((_BN, _H)), _f32((_BN, _W))],
            )(h, aggpre.reshape(_BN, _H), cnt, lp['m2']['w'],
              lp['m2']['b'].reshape(1, _H), lp['u1']['w'],
              lp['u1']['b'].reshape(1, _H), lp['ln']['g'].reshape(1, _H),
              lp['ln']['b'].reshape(1, _H), w1sd(mp[li + 1]))
        else:
            hd = params['head']
            y = pl.pallas_call(
                _final_body,
                out_shape=_f32((_BN, 1)),
            )(h, aggpre.reshape(_BN, _H), cnt, lp['m2']['w'],
              lp['m2']['b'].reshape(1, _H), lp['u1']['w'],
              lp['u1']['b'].reshape(1, _H), lp['ln']['g'].reshape(1, _H),
              lp['ln']['b'].reshape(1, _H), hd['l1']['w'],
              hd['l1']['b'].reshape(1, _H // 2), hd['l2']['w'],
              hd['l2']['b'].reshape(1, 1))

    return y.reshape(_B, _N, 1)
